# baseline (speedup 1.0000x reference)
import functools

import jax
import jax.numpy as jnp
from jax import lax
from jax.experimental import pallas as pl

B, D, K_OLD, K_NEW = 65536, 128, 128, 64
BLK = 1024


def _fused_body(x_ref, w_ref, old_lo_ref, old_hi_ref, new_lo_ref, new_hi_ref,
                out_ref):
    old_lo = old_lo_ref[:]
    old_hi = old_hi_ref[:]
    new_lo = new_lo_ref[:]
    new_hi = new_hi_ref[:]
    overlap = jnp.clip(jnp.minimum(old_hi, new_hi) - jnp.maximum(old_lo, new_lo),
                       0.0, None)
    adaptor = overlap / (old_hi - old_lo)

    logits = jnp.dot(x_ref[:], w_ref[:],
                     preferred_element_type=jnp.float32)
    m = jnp.max(logits, axis=1, keepdims=True)
    e = jnp.exp(logits - m)
    s = jnp.sum(e, axis=1, keepdims=True)
    r = lax.dot_general(e, adaptor, (((1,), (1,)), ((), ())),
                        preferred_element_type=jnp.float32) / s
    out_ref[:] = jnp.log(r + jnp.finfo(jnp.float32).tiny)


@jax.jit
def kernel(x, W, old_edges, new_edges):
    old_lo = old_edges[:-1].reshape(1, K_OLD)
    old_hi = old_edges[1:].reshape(1, K_OLD)
    new_lo = jnp.broadcast_to(new_edges[:-1].reshape(K_NEW, 1), (K_NEW, K_OLD))
    new_hi = jnp.broadcast_to(new_edges[1:].reshape(K_NEW, 1), (K_NEW, K_OLD))

    grid = (B // BLK,)
    return pl.pallas_call(
        _fused_body,
        grid=grid,
        in_specs=[
            pl.BlockSpec((BLK, D), lambda i: (i, 0)),
            pl.BlockSpec((D, K_OLD), lambda i: (0, 0)),
            pl.BlockSpec((1, K_OLD), lambda i: (0, 0)),
            pl.BlockSpec((1, K_OLD), lambda i: (0, 0)),
            pl.BlockSpec((K_NEW, K_OLD), lambda i: (0, 0)),
            pl.BlockSpec((K_NEW, K_OLD), lambda i: (0, 0)),
        ],
        out_specs=pl.BlockSpec((BLK, K_NEW), lambda i: (i, 0)),
        out_shape=jax.ShapeDtypeStruct((B, K_NEW), jnp.float32),
    )(x, W, old_lo, old_hi, new_lo, new_hi)


# speedup vs baseline: 1.4587x; 1.4587x over previous
import functools

import jax
import jax.numpy as jnp
from jax import lax
from jax.experimental import pallas as pl

B, D, K_OLD, K_NEW = 65536, 128, 128, 64
BLK = 4096


def _fused_body(x_ref, w_ref, old_lo_ref, old_hi_ref, new_lo_ref, new_hi_ref,
                out_ref):
    old_lo = old_lo_ref[:]
    old_hi = old_hi_ref[:]
    new_lo = new_lo_ref[:]
    new_hi = new_hi_ref[:]
    overlap = jnp.clip(jnp.minimum(old_hi, new_hi) - jnp.maximum(old_lo, new_lo),
                       0.0, None)
    adaptor = overlap / (old_hi - old_lo)

    logits = jnp.dot(x_ref[:], w_ref[:],
                     preferred_element_type=jnp.float32)
    m = jnp.max(logits, axis=1, keepdims=True)
    e = jnp.exp(logits - m)
    s = jnp.sum(e, axis=1, keepdims=True)
    r = lax.dot_general(e, adaptor, (((1,), (1,)), ((), ())),
                        preferred_element_type=jnp.float32) / s
    out_ref[:] = jnp.log(r + jnp.finfo(jnp.float32).tiny)


@jax.jit
def kernel(x, W, old_edges, new_edges):
    old_lo = old_edges[:-1].reshape(1, K_OLD)
    old_hi = old_edges[1:].reshape(1, K_OLD)
    new_lo = jnp.broadcast_to(new_edges[:-1].reshape(K_NEW, 1), (K_NEW, K_OLD))
    new_hi = jnp.broadcast_to(new_edges[1:].reshape(K_NEW, 1), (K_NEW, K_OLD))

    grid = (B // BLK,)
    return pl.pallas_call(
        _fused_body,
        grid=grid,
        in_specs=[
            pl.BlockSpec((BLK, D), lambda i: (i, 0)),
            pl.BlockSpec((D, K_OLD), lambda i: (0, 0)),
            pl.BlockSpec((1, K_OLD), lambda i: (0, 0)),
            pl.BlockSpec((1, K_OLD), lambda i: (0, 0)),
            pl.BlockSpec((K_NEW, K_OLD), lambda i: (0, 0)),
            pl.BlockSpec((K_NEW, K_OLD), lambda i: (0, 0)),
        ],
        out_specs=pl.BlockSpec((BLK, K_NEW), lambda i: (i, 0)),
        out_shape=jax.ShapeDtypeStruct((B, K_NEW), jnp.float32),
    )(x, W, old_lo, old_hi, new_lo, new_hi)


# speedup vs baseline: 1.5854x; 1.0869x over previous
import functools

import jax
import jax.numpy as jnp
from jax import lax
from jax.experimental import pallas as pl

B, D, K_OLD, K_NEW = 65536, 128, 128, 64
BLK = 8192


def _fused_body(x_ref, w_ref, old_lo_ref, old_hi_ref, new_lo_ref, new_hi_ref,
                out_ref):
    old_lo = old_lo_ref[:]
    old_hi = old_hi_ref[:]
    new_lo = new_lo_ref[:]
    new_hi = new_hi_ref[:]
    overlap = jnp.clip(jnp.minimum(old_hi, new_hi) - jnp.maximum(old_lo, new_lo),
                       0.0, None)
    adaptor = overlap / (old_hi - old_lo)

    logits = jnp.dot(x_ref[:], w_ref[:],
                     preferred_element_type=jnp.float32)
    m = jnp.max(logits, axis=1, keepdims=True)
    e = jnp.exp(logits - m)
    s = jnp.sum(e, axis=1, keepdims=True)
    r = lax.dot_general(e, adaptor, (((1,), (1,)), ((), ())),
                        preferred_element_type=jnp.float32) / s
    out_ref[:] = jnp.log(r + jnp.finfo(jnp.float32).tiny)


@jax.jit
def kernel(x, W, old_edges, new_edges):
    old_lo = old_edges[:-1].reshape(1, K_OLD)
    old_hi = old_edges[1:].reshape(1, K_OLD)
    new_lo = jnp.broadcast_to(new_edges[:-1].reshape(K_NEW, 1), (K_NEW, K_OLD))
    new_hi = jnp.broadcast_to(new_edges[1:].reshape(K_NEW, 1), (K_NEW, K_OLD))

    grid = (B // BLK,)
    return pl.pallas_call(
        _fused_body,
        grid=grid,
        in_specs=[
            pl.BlockSpec((BLK, D), lambda i: (i, 0)),
            pl.BlockSpec((D, K_OLD), lambda i: (0, 0)),
            pl.BlockSpec((1, K_OLD), lambda i: (0, 0)),
            pl.BlockSpec((1, K_OLD), lambda i: (0, 0)),
            pl.BlockSpec((K_NEW, K_OLD), lambda i: (0, 0)),
            pl.BlockSpec((K_NEW, K_OLD), lambda i: (0, 0)),
        ],
        out_specs=pl.BlockSpec((BLK, K_NEW), lambda i: (i, 0)),
        out_shape=jax.ShapeDtypeStruct((B, K_NEW), jnp.float32),
    )(x, W, old_lo, old_hi, new_lo, new_hi)


# speedup vs baseline: 1.5989x; 1.0085x over previous
import functools

import jax
import jax.numpy as jnp
from jax import lax
from jax.experimental import pallas as pl

B, D, K_OLD, K_NEW = 65536, 128, 128, 64
BLK = 16384


def _fused_body(x_ref, w_ref, old_lo_ref, old_hi_ref, new_lo_ref, new_hi_ref,
                out_ref):
    old_lo = old_lo_ref[:]
    old_hi = old_hi_ref[:]
    new_lo = new_lo_ref[:]
    new_hi = new_hi_ref[:]
    overlap = jnp.clip(jnp.minimum(old_hi, new_hi) - jnp.maximum(old_lo, new_lo),
                       0.0, None)
    adaptor = overlap / (old_hi - old_lo)

    logits = jnp.dot(x_ref[:], w_ref[:],
                     preferred_element_type=jnp.float32)
    m = jnp.max(logits, axis=1, keepdims=True)
    e = jnp.exp(logits - m)
    s = jnp.sum(e, axis=1, keepdims=True)
    r = lax.dot_general(e, adaptor, (((1,), (1,)), ((), ())),
                        preferred_element_type=jnp.float32) / s
    out_ref[:] = jnp.log(r + jnp.finfo(jnp.float32).tiny)


@jax.jit
def kernel(x, W, old_edges, new_edges):
    old_lo = old_edges[:-1].reshape(1, K_OLD)
    old_hi = old_edges[1:].reshape(1, K_OLD)
    new_lo = jnp.broadcast_to(new_edges[:-1].reshape(K_NEW, 1), (K_NEW, K_OLD))
    new_hi = jnp.broadcast_to(new_edges[1:].reshape(K_NEW, 1), (K_NEW, K_OLD))

    grid = (B // BLK,)
    return pl.pallas_call(
        _fused_body,
        grid=grid,
        in_specs=[
            pl.BlockSpec((BLK, D), lambda i: (i, 0)),
            pl.BlockSpec((D, K_OLD), lambda i: (0, 0)),
            pl.BlockSpec((1, K_OLD), lambda i: (0, 0)),
            pl.BlockSpec((1, K_OLD), lambda i: (0, 0)),
            pl.BlockSpec((K_NEW, K_OLD), lambda i: (0, 0)),
            pl.BlockSpec((K_NEW, K_OLD), lambda i: (0, 0)),
        ],
        out_specs=pl.BlockSpec((BLK, K_NEW), lambda i: (i, 0)),
        out_shape=jax.ShapeDtypeStruct((B, K_NEW), jnp.float32),
    )(x, W, old_lo, old_hi, new_lo, new_hi)
